# baseline (speedup 1.0000x reference)
import functools
import jax
import jax.numpy as jnp
from jax import lax
from jax.experimental import pallas as pl
from jax.experimental.pallas import tpu as pltpu
from jax.experimental.pallas import tpu_sc as plsc

_D = 32
_B = 16384
_NC = 2
_NS = 16
_NW = _NC * _NS
_BPW = _B // _NW
_CHUNK = 128
_NCHUNK = _BPW // _CHUNK


def _emb_kernel(idx_hbm, table_hbm, out_hbm, idx_v, rows_v, sems):
    wid = lax.axis_index("s") * _NC + lax.axis_index("c")
    base = wid * _BPW
    pltpu.sync_copy(idx_hbm.at[wid], idx_v)
    copies = []
    for j in range(_NCHUNK):
        copies.append(
            pltpu.async_copy(
                table_hbm.at[idx_v.at[j]],
                rows_v.at[pl.ds(j * _CHUNK, _CHUNK)],
                sems.at[j],
            )
        )
    for c in copies:
        c.wait()
    pltpu.sync_copy(rows_v, out_hbm.at[pl.ds(base, _BPW)])


@jax.jit
def _embedding_lookup(x, params):
    idx = x.astype(jnp.int32).reshape(_NW, _NCHUNK, _CHUNK)
    run = functools.partial(
        pl.kernel,
        mesh=plsc.VectorSubcoreMesh(core_axis_name="c", subcore_axis_name="s"),
        out_type=jax.ShapeDtypeStruct((_B, _D), jnp.float32),
        scratch_types=[
            pltpu.VMEM((_NCHUNK, _CHUNK), jnp.int32),
            pltpu.VMEM((_BPW, _D), jnp.float32),
            pltpu.SemaphoreType.DMA((_NCHUNK,)),
        ],
        compiler_params=pltpu.CompilerParams(use_tc_tiling_on_sc=False),
    )(_emb_kernel)
    return run(idx, params)


def kernel(x, params):
    return _embedding_lookup(x, params)


# speedup vs baseline: 1.0003x; 1.0003x over previous
import functools
import jax
import jax.numpy as jnp
from jax import lax
from jax.experimental import pallas as pl
from jax.experimental.pallas import tpu as pltpu
from jax.experimental.pallas import tpu_sc as plsc

_V = 1000000
_D = 32
_B = 16384
_NC = 2
_NS = 16
_NW = _NC * _NS
_BPW = _B // _NW
_CHUNK = 128
_NCHUNK = _BPW // _CHUNK


def _emb_kernel(idx_hbm, table_hbm, out_hbm, idx_v, rows_v, sems):
    wid = lax.axis_index("s") * _NC + lax.axis_index("c")
    base = wid * _BPW
    pltpu.sync_copy(idx_hbm.at[wid], idx_v)
    copies = []
    for j in range(_NCHUNK):
        copies.append(
            pltpu.async_copy(
                table_hbm.at[idx_v.at[j]],
                rows_v.at[pl.ds(j * _CHUNK, _CHUNK)],
                sems.at[j],
            )
        )
    for c in copies:
        c.wait()
    pltpu.sync_copy(rows_v, out_hbm.at[pl.ds(base, _BPW)])


@jax.jit
def _embedding_lookup(x, params):
    lin = jax.lax.optimization_barrier(params.reshape(_V * _D))
    table = lin.reshape(_V, _D)
    idx = x.astype(jnp.int32).reshape(_NW, _NCHUNK, _CHUNK)
    run = functools.partial(
        pl.kernel,
        mesh=plsc.VectorSubcoreMesh(core_axis_name="c", subcore_axis_name="s"),
        out_type=jax.ShapeDtypeStruct((_B, _D), jnp.float32),
        scratch_types=[
            pltpu.VMEM((_NCHUNK, _CHUNK), jnp.int32),
            pltpu.VMEM((_BPW, _D), jnp.float32),
            pltpu.SemaphoreType.DMA((_NCHUNK,)),
        ],
        compiler_params=pltpu.CompilerParams(use_tc_tiling_on_sc=False),
    )(_emb_kernel)
    return run(idx, table)


def kernel(x, params):
    return _embedding_lookup(x, params)


# speedup vs baseline: 2.4423x; 2.4415x over previous
import functools
import jax
import jax.numpy as jnp
from jax import lax
from jax.experimental import pallas as pl
from jax.experimental.pallas import tpu as pltpu
from jax.experimental.pallas import tpu_sc as plsc

_V = 1000000
_D = 32
_B = 16384
_NC = 2
_NS = 16
_NW = _NC * _NS
_BPW = _B // _NW
_LANE = 16

_K1W = 1536
_LPT = 31232
_NWIN = 20
_TAIL0 = _NW * _LPT
_CHUNK = 128


_NCOLS = 7812
_CGRP = 12
_ROWS = _NCOLS * _D
_RSTRIDE = _NCOLS * 1024


def _detile_kernel(table_hbm, lin_hbm, buf, rsem, wsem):
    wid = lax.axis_index("s") * _NC + lax.axis_index("c")
    r = wid // 8
    k = wid % 8
    ncols = _NCOLS // 8
    c0 = k * ncols

    def grp_copy(cbase, ngrp, carry):
        copies = []
        for u in range(ngrp):
            copies.append(
                pltpu.async_copy(
                    table_hbm.at[r, :, pl.ds((cbase + u) * 128, 128)],
                    buf.at[pl.ds(u * 8, 8), :],
                    rsem,
                )
            )
        for c in copies:
            c.wait()
        row0 = (r * _NCOLS + cbase) * 8
        pltpu.async_copy(
            buf.at[pl.ds(0, ngrp * 8), :],
            lin_hbm.at[pl.ds(row0, ngrp * 8), :],
            wsem,
        ).wait()
        return carry

    def body(t, carry):
        return grp_copy(c0 + t * _CGRP, _CGRP, carry)

    lax.fori_loop(0, ncols // _CGRP, body, 0)
    grp_copy(c0 + (ncols // _CGRP) * _CGRP, ncols % _CGRP, 0)
    @pl.when(k == 0)
    def _():
        grp_copy(8 * ncols, _NCOLS - 8 * ncols, 0)


_TBASE = _V - 64


def _gather_kernel(x_hbm, lin_hbm, tail_hbm, out_hbm, idx_v, off_v, rows_v, tail_v, sems):
    wid = lax.axis_index("s") * _NC + lax.axis_index("c")
    base = wid * _BPW
    pltpu.sync_copy(x_hbm.at[pl.ds(base, _BPW)], idx_v)
    pltpu.sync_copy(tail_hbm, tail_v)

    def offsets(j, carry):
        c = (j // 8) * _RSTRIDE + (j % 8) * 128
        for v in range(_BPW // _LANE):
            xi = idx_v[pl.ds(v * _LANE, _LANE)]
            b = ((xi >> 7) << 10) | (xi & 127)
            off_v[j, pl.ds(v * _LANE, _LANE)] = b + c
        return carry

    lax.fori_loop(0, _D, offsets, 0)

    def gather(g, carry):
        copies = []
        for u in range(8):
            k = g * 8 + u
            j = k // 4
            p0 = (k % 4) * _CHUNK
            copies.append(
                pltpu.async_copy(
                    lin_hbm.at[off_v.at[j, pl.ds(p0, _CHUNK)]],
                    rows_v.at[j, pl.ds(p0, _CHUNK)],
                    sems.at[u],
                )
            )
        for c in copies:
            c.wait()
        return carry

    lax.fori_loop(0, (_D * _BPW // _CHUNK) // 8, gather, 0)

    def patch(v, carry):
        xi = idx_v[pl.ds(v * _LANE, _LANE)]
        m = xi >= _TBASE
        t = jnp.maximum(xi - _TBASE, 0)
        pidx = lax.iota(jnp.int32, _LANE) + v * _LANE
        for j in range(_D):
            js = jnp.full((_LANE,), j, jnp.int32)
            val = plsc.load_gather(tail_v, [t, js])
            plsc.store_scatter(rows_v, [js, pidx], val, mask=m)
        return carry

    lax.fori_loop(0, _BPW // _LANE, patch, 0)
    pltpu.sync_copy(rows_v, out_hbm.at[:, pl.ds(base, _BPW)])


@jax.jit
def _embedding_lookup(x, params):
    mesh = plsc.VectorSubcoreMesh(core_axis_name="c", subcore_axis_name="s")
    detile = functools.partial(
        pl.kernel,
        mesh=mesh,
        out_type=jax.ShapeDtypeStruct((_ROWS, 128), jnp.float32),
        scratch_types=[
            pltpu.VMEM((_CGRP * 8, 128), jnp.float32),
            pltpu.SemaphoreType.DMA,
            pltpu.SemaphoreType.DMA,
        ],
    )(_detile_kernel)
    gather = functools.partial(
        pl.kernel,
        mesh=mesh,
        out_type=jax.ShapeDtypeStruct((_D, _B), jnp.float32),
        scratch_types=[
            pltpu.VMEM((_BPW,), jnp.int32),
            pltpu.VMEM((_D, _BPW), jnp.int32),
            pltpu.VMEM((_D, _BPW), jnp.float32),
            pltpu.VMEM((64, _D), jnp.float32),
            pltpu.SemaphoreType.DMA((8,)),
        ],
        compiler_params=pltpu.CompilerParams(
            use_tc_tiling_on_sc=False, needs_layout_passes=False
        ),
    )(_gather_kernel)
    lin2 = detile(params.T.reshape(_D // 8, 8, _V))
    lin = lin2.reshape(_ROWS * 128)
    tail = params[_TBASE:, :]
    out_t = gather(x.astype(jnp.int32), lin, tail)
    return out_t.T


def kernel(x, params):
    return _embedding_lookup(x, params)


# speedup vs baseline: 3.4179x; 1.3994x over previous
import functools
import jax
import jax.numpy as jnp
from jax import lax
from jax.experimental import pallas as pl
from jax.experimental.pallas import tpu as pltpu
from jax.experimental.pallas import tpu_sc as plsc

_V = 1000000
_D = 32
_B = 16384
_NC = 2
_NS = 16
_NW = _NC * _NS
_BPW = _B // _NW
_LANE = 16

_K1W = 1536
_LPT = 31232
_NWIN = 20
_TAIL0 = _NW * _LPT
_CHUNK = 128


_NCOLS = 7812
_CGRP = 16
_ROWS = _NCOLS * _D
_RSTRIDE = _NCOLS * 1024


def _detile_kernel(table_hbm, lin_hbm, buf, dummy_v, rsems, wsems):
    wid = lax.axis_index("s") * _NC + lax.axis_index("c")
    r = wid // 8
    k = wid % 8
    ncols = _NCOLS // 8
    c0 = k * ncols
    nb = ncols // _CGRP

    def fire_reads(t, b):
        cbase = c0 + t * _CGRP
        for u in range(_CGRP):
            pltpu.async_copy(
                table_hbm.at[r, :, pl.ds((cbase + u) * 128, 128)],
                buf.at[b, pl.ds(u * 8, 8), :],
                rsems.at[b],
            )

    def fire_write(t, b):
        row0 = (r * _NCOLS + c0 + t * _CGRP) * 8
        pltpu.async_copy(
            buf.at[b],
            lin_hbm.at[pl.ds(row0, _CGRP * 8), :],
            wsems.at[b],
        )

    def drain(sem):
        pltpu.make_async_copy(
            lin_hbm.at[pl.ds(0, _CGRP * 8), :], dummy_v, sem
        ).wait()

    def body(t, carry):
        b = t % 2
        @pl.when(t >= 1)
        def _():
            drain(wsems.at[1 - b])
        @pl.when(t + 1 < nb)
        def _():
            fire_reads(t + 1, 1 - b)
        drain(rsems.at[b])
        fire_write(t, b)
        return carry

    fire_reads(0, 0)
    lax.fori_loop(0, nb, body, 0)
    drain(wsems.at[(nb - 1) % 2])

    @pl.when(k == 0)
    def _():
        nleft = _NCOLS - 8 * ncols
        copies = []
        for u in range(nleft):
            copies.append(
                pltpu.async_copy(
                    table_hbm.at[r, :, pl.ds((8 * ncols + u) * 128, 128)],
                    buf.at[0, pl.ds(u * 8, 8), :],
                    rsems.at[0],
                )
            )
        for c in copies:
            c.wait()
        pltpu.async_copy(
            buf.at[0, pl.ds(0, nleft * 8), :],
            lin_hbm.at[pl.ds((r * _NCOLS + 8 * ncols) * 8, nleft * 8), :],
            wsems.at[0],
        ).wait()


_TBASE = _V - 64


def _gather_kernel(x_hbm, lin_hbm, tail_hbm, out_hbm, idx_v, off_v, rows_v, tail_v, sems):
    wid = lax.axis_index("s") * _NC + lax.axis_index("c")
    base = wid * _BPW
    pltpu.sync_copy(x_hbm.at[pl.ds(base, _BPW)], idx_v)
    pltpu.sync_copy(tail_hbm, tail_v)

    def offsets(j, carry):
        c = (j // 8) * _RSTRIDE + (j % 8) * 128
        for v in range(_BPW // _LANE):
            xi = idx_v[pl.ds(v * _LANE, _LANE)]
            b = ((xi >> 7) << 10) | (xi & 127)
            off_v[j, pl.ds(v * _LANE, _LANE)] = b + c
        return carry

    lax.fori_loop(0, _D, offsets, 0)

    def gather(g, carry):
        copies = []
        for u in range(8):
            k = g * 8 + u
            j = k // 4
            p0 = (k % 4) * _CHUNK
            copies.append(
                pltpu.async_copy(
                    lin_hbm.at[off_v.at[j, pl.ds(p0, _CHUNK)]],
                    rows_v.at[j, pl.ds(p0, _CHUNK)],
                    sems.at[u],
                )
            )
        for c in copies:
            c.wait()
        return carry

    lax.fori_loop(0, (_D * _BPW // _CHUNK) // 8, gather, 0)

    def patch(v, carry):
        xi = idx_v[pl.ds(v * _LANE, _LANE)]
        m = xi >= _TBASE
        t = jnp.maximum(xi - _TBASE, 0)
        pidx = lax.iota(jnp.int32, _LANE) + v * _LANE
        for j in range(_D):
            js = jnp.full((_LANE,), j, jnp.int32)
            val = plsc.load_gather(tail_v, [t, js])
            plsc.store_scatter(rows_v, [js, pidx], val, mask=m)
        return carry

    lax.fori_loop(0, _BPW // _LANE, patch, 0)
    pltpu.sync_copy(rows_v, out_hbm.at[:, pl.ds(base, _BPW)])


@jax.jit
def _embedding_lookup(x, params):
    mesh = plsc.VectorSubcoreMesh(core_axis_name="c", subcore_axis_name="s")
    detile = functools.partial(
        pl.kernel,
        mesh=mesh,
        out_type=jax.ShapeDtypeStruct((_ROWS, 128), jnp.float32),
        scratch_types=[
            pltpu.VMEM((2, _CGRP * 8, 128), jnp.float32),
            pltpu.VMEM((_CGRP * 8, 128), jnp.float32),
            pltpu.SemaphoreType.DMA((2,)),
            pltpu.SemaphoreType.DMA((2,)),
        ],
    )(_detile_kernel)
    gather = functools.partial(
        pl.kernel,
        mesh=mesh,
        out_type=jax.ShapeDtypeStruct((_D, _B), jnp.float32),
        scratch_types=[
            pltpu.VMEM((_BPW,), jnp.int32),
            pltpu.VMEM((_D, _BPW), jnp.int32),
            pltpu.VMEM((_D, _BPW), jnp.float32),
            pltpu.VMEM((64, _D), jnp.float32),
            pltpu.SemaphoreType.DMA((8,)),
        ],
        compiler_params=pltpu.CompilerParams(
            use_tc_tiling_on_sc=False, needs_layout_passes=False
        ),
    )(_gather_kernel)
    lin2 = detile(params.T.reshape(_D // 8, 8, _V))
    lin = lin2.reshape(_ROWS * 128)
    tail = params[_TBASE:, :]
    out_t = gather(x.astype(jnp.int32), lin, tail)
    return out_t.T


def kernel(x, params):
    return _embedding_lookup(x, params)


# speedup vs baseline: 3.5103x; 1.0271x over previous
import functools
import jax
import jax.numpy as jnp
from jax import lax
from jax.experimental import pallas as pl
from jax.experimental.pallas import tpu as pltpu
from jax.experimental.pallas import tpu_sc as plsc

_V = 1000000
_D = 32
_B = 16384
_NC = 2
_NS = 16
_NW = _NC * _NS
_BPW = _B // _NW
_LANE = 16

_K1W = 1536
_LPT = 31232
_NWIN = 20
_TAIL0 = _NW * _LPT
_CHUNK = 128


_NCOLS = 7812
_CGRP = 16
_ROWS = _NCOLS * _D
_RSTRIDE = _NCOLS * 1024


def _detile_kernel(table_hbm, lin_hbm, buf, dummy_v, rsems, wsems):
    wid = lax.axis_index("s") * _NC + lax.axis_index("c")
    r = wid // 8
    k = wid % 8
    ncols = _NCOLS // 8
    c0 = k * ncols
    nb = ncols // _CGRP

    def fire_reads(t, b):
        cbase = c0 + t * _CGRP
        for u in range(_CGRP):
            pltpu.async_copy(
                table_hbm.at[r, :, pl.ds((cbase + u) * 128, 128)],
                buf.at[b, pl.ds(u * 8, 8), :],
                rsems.at[b],
            )

    def fire_write(t, b):
        row0 = (r * _NCOLS + c0 + t * _CGRP) * 8
        pltpu.async_copy(
            buf.at[b],
            lin_hbm.at[pl.ds(row0, _CGRP * 8), :],
            wsems.at[b],
        )

    def drain(sem):
        pltpu.make_async_copy(
            lin_hbm.at[pl.ds(0, _CGRP * 8), :], dummy_v, sem
        ).wait()

    def body(t, carry):
        b = t % 2
        @pl.when(t >= 1)
        def _():
            drain(wsems.at[1 - b])
        @pl.when(t + 1 < nb)
        def _():
            fire_reads(t + 1, 1 - b)
        drain(rsems.at[b])
        fire_write(t, b)
        return carry

    fire_reads(0, 0)
    lax.fori_loop(0, nb, body, 0)
    drain(wsems.at[(nb - 1) % 2])

    @pl.when(k == 0)
    def _():
        nleft = _NCOLS - 8 * ncols
        copies = []
        for u in range(nleft):
            copies.append(
                pltpu.async_copy(
                    table_hbm.at[r, :, pl.ds((8 * ncols + u) * 128, 128)],
                    buf.at[0, pl.ds(u * 8, 8), :],
                    rsems.at[0],
                )
            )
        for c in copies:
            c.wait()
        pltpu.async_copy(
            buf.at[0, pl.ds(0, nleft * 8), :],
            lin_hbm.at[pl.ds((r * _NCOLS + 8 * ncols) * 8, nleft * 8), :],
            wsems.at[0],
        ).wait()


_TBASE = _V - 64


def _gather_kernel(x_hbm, lin_hbm, tail_hbm, out_hbm, idx_v, off_v, rows_v, tail_v, gdummy_v, sems):
    wid = lax.axis_index("s") * _NC + lax.axis_index("c")
    base = wid * _BPW
    pltpu.sync_copy(x_hbm.at[pl.ds(base, _BPW)], idx_v)
    pltpu.sync_copy(tail_hbm, tail_v)

    def offsets(j, carry):
        c = (j // 8) * _RSTRIDE + (j % 8) * 128
        for v in range(_BPW // _LANE):
            xi = idx_v[pl.ds(v * _LANE, _LANE)]
            b = ((xi >> 7) << 10) | (xi & 127)
            off_v[j, pl.ds(v * _LANE, _LANE)] = b + c
        return carry

    lax.fori_loop(0, _D, offsets, 0)

    def gdrain(sem):
        pltpu.make_async_copy(
            lin_hbm.at[pl.ds(0, 8 * _CHUNK)], gdummy_v, sem
        ).wait()

    def gather(g, carry):
        b = g % 2
        @pl.when(g >= 2)
        def _():
            gdrain(sems.at[b])
        for u in range(8):
            k = g * 8 + u
            j = k // 4
            p0 = (k % 4) * _CHUNK
            pltpu.async_copy(
                lin_hbm.at[off_v.at[j, pl.ds(p0, _CHUNK)]],
                rows_v.at[j, pl.ds(p0, _CHUNK)],
                sems.at[b],
            )
        return carry

    nbat = (_D * _BPW // _CHUNK) // 8
    lax.fori_loop(0, nbat, gather, 0)
    gdrain(sems.at[0])
    gdrain(sems.at[1])

    def patch(v, carry):
        xi = idx_v[pl.ds(v * _LANE, _LANE)]
        m = xi >= _TBASE
        t = jnp.maximum(xi - _TBASE, 0)
        pidx = lax.iota(jnp.int32, _LANE) + v * _LANE
        for j in range(_D):
            js = jnp.full((_LANE,), j, jnp.int32)
            val = plsc.load_gather(tail_v, [t, js])
            plsc.store_scatter(rows_v, [js, pidx], val, mask=m)
        return carry

    lax.fori_loop(0, _BPW // _LANE, patch, 0)
    pltpu.sync_copy(rows_v, out_hbm.at[:, pl.ds(base, _BPW)])


@jax.jit
def _embedding_lookup(x, params):
    mesh = plsc.VectorSubcoreMesh(core_axis_name="c", subcore_axis_name="s")
    detile = functools.partial(
        pl.kernel,
        mesh=mesh,
        out_type=jax.ShapeDtypeStruct((_ROWS, 128), jnp.float32),
        scratch_types=[
            pltpu.VMEM((2, _CGRP * 8, 128), jnp.float32),
            pltpu.VMEM((_CGRP * 8, 128), jnp.float32),
            pltpu.SemaphoreType.DMA((2,)),
            pltpu.SemaphoreType.DMA((2,)),
        ],
    )(_detile_kernel)
    gather = functools.partial(
        pl.kernel,
        mesh=mesh,
        out_type=jax.ShapeDtypeStruct((_D, _B), jnp.float32),
        scratch_types=[
            pltpu.VMEM((_BPW,), jnp.int32),
            pltpu.VMEM((_D, _BPW), jnp.int32),
            pltpu.VMEM((_D, _BPW), jnp.float32),
            pltpu.VMEM((64, _D), jnp.float32),
            pltpu.VMEM((8 * _CHUNK,), jnp.float32),
            pltpu.SemaphoreType.DMA((2,)),
        ],
        compiler_params=pltpu.CompilerParams(
            use_tc_tiling_on_sc=False, needs_layout_passes=False
        ),
    )(_gather_kernel)
    lin2 = detile(params.T.reshape(_D // 8, 8, _V))
    lin = lin2.reshape(_ROWS * 128)
    tail = params[_TBASE:, :]
    out_t = gather(x.astype(jnp.int32), lin, tail)
    return out_t.T


def kernel(x, params):
    return _embedding_lookup(x, params)


# speedup vs baseline: 3.5383x; 1.0080x over previous
import functools
import jax
import jax.numpy as jnp
from jax import lax
from jax.experimental import pallas as pl
from jax.experimental.pallas import tpu as pltpu
from jax.experimental.pallas import tpu_sc as plsc

_V = 1000000
_D = 32
_B = 16384
_NC = 2
_NS = 16
_NW = _NC * _NS
_BPW = _B // _NW
_LANE = 16

_K1W = 1536
_LPT = 31232
_NWIN = 20
_TAIL0 = _NW * _LPT
_CHUNK = 128


_NCOLS = 7812
_CGRP = 16
_ROWS = _NCOLS * _D
_RSTRIDE = _NCOLS * 1024


def _detile_kernel(table_hbm, lin_hbm, buf, dummy_v, rsems, wsems):
    wid = lax.axis_index("s") * _NC + lax.axis_index("c")
    r = wid // 8
    k = wid % 8
    ncols = _NCOLS // 8
    c0 = k * ncols
    nb = ncols // _CGRP

    def fire_reads(t, b):
        cbase = c0 + t * _CGRP
        for u in range(_CGRP):
            pltpu.async_copy(
                table_hbm.at[r, :, pl.ds((cbase + u) * 128, 128)],
                buf.at[b, pl.ds(u * 8, 8), :],
                rsems.at[b],
            )

    def fire_write(t, b):
        row0 = (r * _NCOLS + c0 + t * _CGRP) * 8
        pltpu.async_copy(
            buf.at[b],
            lin_hbm.at[pl.ds(row0, _CGRP * 8), :],
            wsems.at[b],
        )

    def drain(sem):
        pltpu.make_async_copy(
            lin_hbm.at[pl.ds(0, _CGRP * 8), :], dummy_v, sem
        ).wait()

    def body(t, carry):
        b = t % 3
        bn = (t + 2) % 3
        @pl.when(t >= 1)
        def _():
            drain(wsems.at[bn])
        @pl.when(t + 2 < nb)
        def _():
            fire_reads(t + 2, bn)
        drain(rsems.at[b])
        fire_write(t, b)
        return carry

    fire_reads(0, 0)
    fire_reads(1, 1)
    lax.fori_loop(0, nb, body, 0)
    drain(wsems.at[(nb - 1) % 3])

    @pl.when(k == 0)
    def _():
        nleft = _NCOLS - 8 * ncols
        copies = []
        for u in range(nleft):
            copies.append(
                pltpu.async_copy(
                    table_hbm.at[r, :, pl.ds((8 * ncols + u) * 128, 128)],
                    buf.at[0, pl.ds(u * 8, 8), :],
                    rsems.at[0],
                )
            )
        for c in copies:
            c.wait()
        pltpu.async_copy(
            buf.at[0, pl.ds(0, nleft * 8), :],
            lin_hbm.at[pl.ds((r * _NCOLS + 8 * ncols) * 8, nleft * 8), :],
            wsems.at[0],
        ).wait()


_TBASE = _V - 64


def _gather_kernel(x_hbm, lin_hbm, tail_hbm, out_hbm, idx_v, off_v, rows_v, tail_v, gdummy_v, sems):
    wid = lax.axis_index("s") * _NC + lax.axis_index("c")
    base = wid * _BPW
    pltpu.sync_copy(x_hbm.at[pl.ds(base, _BPW)], idx_v)
    pltpu.sync_copy(tail_hbm, tail_v)

    def offsets(j, carry):
        c = (j // 8) * _RSTRIDE + (j % 8) * 128
        for v in range(_BPW // _LANE):
            xi = idx_v[pl.ds(v * _LANE, _LANE)]
            b = ((xi >> 7) << 10) | (xi & 127)
            off_v[j, pl.ds(v * _LANE, _LANE)] = b + c
        return carry

    lax.fori_loop(0, _D, offsets, 0)

    def gdrain(sem):
        pltpu.make_async_copy(
            lin_hbm.at[pl.ds(0, 8 * _CHUNK)], gdummy_v, sem
        ).wait()

    def gather(g, carry):
        b = g % 2
        @pl.when(g >= 2)
        def _():
            gdrain(sems.at[b])
        for u in range(8):
            k = g * 8 + u
            j = k // 4
            p0 = (k % 4) * _CHUNK
            pltpu.async_copy(
                lin_hbm.at[off_v.at[j, pl.ds(p0, _CHUNK)]],
                rows_v.at[j, pl.ds(p0, _CHUNK)],
                sems.at[b],
            )
        return carry

    nbat = (_D * _BPW // _CHUNK) // 8
    lax.fori_loop(0, nbat, gather, 0)
    gdrain(sems.at[0])
    gdrain(sems.at[1])

    def patch(v, carry):
        xi = idx_v[pl.ds(v * _LANE, _LANE)]
        m = xi >= _TBASE
        t = jnp.maximum(xi - _TBASE, 0)
        pidx = lax.iota(jnp.int32, _LANE) + v * _LANE
        for j in range(_D):
            js = jnp.full((_LANE,), j, jnp.int32)
            val = plsc.load_gather(tail_v, [t, js])
            plsc.store_scatter(rows_v, [js, pidx], val, mask=m)
        return carry

    lax.fori_loop(0, _BPW // _LANE, patch, 0)
    pltpu.sync_copy(rows_v, out_hbm.at[:, pl.ds(base, _BPW)])


@jax.jit
def _embedding_lookup(x, params):
    mesh = plsc.VectorSubcoreMesh(core_axis_name="c", subcore_axis_name="s")
    detile = functools.partial(
        pl.kernel,
        mesh=mesh,
        out_type=jax.ShapeDtypeStruct((_ROWS, 128), jnp.float32),
        scratch_types=[
            pltpu.VMEM((3, _CGRP * 8, 128), jnp.float32),
            pltpu.VMEM((_CGRP * 8, 128), jnp.float32),
            pltpu.SemaphoreType.DMA((3,)),
            pltpu.SemaphoreType.DMA((3,)),
        ],
    )(_detile_kernel)
    gather = functools.partial(
        pl.kernel,
        mesh=mesh,
        out_type=jax.ShapeDtypeStruct((_D, _B), jnp.float32),
        scratch_types=[
            pltpu.VMEM((_BPW,), jnp.int32),
            pltpu.VMEM((_D, _BPW), jnp.int32),
            pltpu.VMEM((_D, _BPW), jnp.float32),
            pltpu.VMEM((64, _D), jnp.float32),
            pltpu.VMEM((8 * _CHUNK,), jnp.float32),
            pltpu.SemaphoreType.DMA((2,)),
        ],
        compiler_params=pltpu.CompilerParams(
            use_tc_tiling_on_sc=False, needs_layout_passes=False
        ),
    )(_gather_kernel)
    lin2 = detile(params.T.reshape(_D // 8, 8, _V))
    lin = lin2.reshape(_ROWS * 128)
    tail = params[_TBASE:, :]
    out_t = gather(x.astype(jnp.int32), lin, tail)
    return out_t.T


def kernel(x, params):
    return _embedding_lookup(x, params)
